# baseline (speedup 1.0000x reference)
import functools

import jax
import jax.numpy as jnp
from jax import lax
from jax.experimental import pallas as pl
from jax.experimental.pallas import tpu as pltpu
from jax.experimental.pallas import tpu_sc as plsc

_NC = 2
_NS = 16
_NW = _NC * _NS

_U_ROWS = 1000000
_I_ROWS = 100000
_U_SHARD = 31256
_I_SHARD = 3128

_mesh = plsc.VectorSubcoreMesh(core_axis_name="c", subcore_axis_name="s")


def _shard_copy(src, dst, shard, total, wid):
    off = wid * shard
    off = jnp.minimum(off, total - shard)
    pltpu.sync_copy(src.at[pl.ds(off, shard)], dst.at[pl.ds(off, shard)])


@functools.partial(
    pl.kernel,
    out_type=[
        jax.ShapeDtypeStruct((_U_ROWS, 32), jnp.float32),
        jax.ShapeDtypeStruct((_I_ROWS, 32), jnp.float32),
    ],
    mesh=_mesh,
)
def _sc_copy(u_in, i_in, u_out, i_out):
    wid = lax.axis_index("s") * _NC + lax.axis_index("c")
    _shard_copy(u_in, u_out, _U_SHARD, _U_ROWS, wid)
    _shard_copy(i_in, i_out, _I_SHARD, _I_ROWS, wid)


def kernel(emb_user, emb_item):
    u, i = _sc_copy(emb_user, emb_item)
    return (u, i)


# speedup vs baseline: 16.8702x; 16.8702x over previous
import functools

import jax
import jax.numpy as jnp
from jax import lax
from jax.experimental import pallas as pl
from jax.experimental.pallas import tpu as pltpu
from jax.experimental.pallas import tpu_sc as plsc

_NC = 2
_NS = 16
_NW = _NC * _NS

_C = 1000
_U_ROWS = 1000000
_I_ROWS = 100000
_UG = _U_ROWS // _C
_IG = _I_ROWS // _C
_UJ = -(-_UG // _NW)
_IJ = -(-_IG // _NW)

_mesh = plsc.VectorSubcoreMesh(core_axis_name="c", subcore_axis_name="s")


@functools.partial(
    pl.kernel,
    out_type=[
        jax.ShapeDtypeStruct((_U_ROWS, 32), jnp.float32),
        jax.ShapeDtypeStruct((_I_ROWS, 32), jnp.float32),
    ],
    mesh=_mesh,
    scratch_types=[pltpu.VMEM((_C, 32), jnp.float32)],
)
def _sc_copy(u_in, i_in, u_out, i_out, buf):
    wid = lax.axis_index("s") * _NC + lax.axis_index("c")

    for j in range(_UJ):
        k = wid * _UJ + j

        @pl.when(k < _UG)
        def _():
            off = k * _C
            pltpu.sync_copy(u_in.at[pl.ds(off, _C)], buf)
            pltpu.sync_copy(buf, u_out.at[pl.ds(off, _C)])

    for j in range(_IJ):
        k = wid * _IJ + j

        @pl.when(k < _IG)
        def _():
            off = k * _C
            pltpu.sync_copy(i_in.at[pl.ds(off, _C)], buf)
            pltpu.sync_copy(buf, i_out.at[pl.ds(off, _C)])


def kernel(emb_user, emb_item):
    u, i = _sc_copy(emb_user, emb_item)
    return (u, i)
